# Initial kernel scaffold; baseline (speedup 1.0000x reference)
import jax
import jax.numpy as jnp
from jax.experimental import pallas as pl


def kernel(x, edge_index, W1, b1, W2, b2, Wl, bl):
    raise NotImplementedError("write your pallas kernel here")


# speedup vs baseline: 23.0945x; 23.0945x over previous
import functools

import jax
import jax.numpy as jnp
from jax import lax
from jax.experimental import pallas as pl
from jax.experimental.pallas import tpu as pltpu
from jax.experimental.pallas import tpu_sc as plsc

N = 10000
E = 320000
IN_CH = 128
HID = 64
OUT_CH = 2

NC = 2
NS = 16
NW = NC * NS
EPT = E // NW
K = 80
NCH = EPT // K
RPT = 624
TAIL = N - NS * RPT
TAIL_OFF = NS * RPT

_mesh = plsc.VectorSubcoreMesh(core_axis_name="c", subcore_axis_name="s")


def _striped(s, copy_fn):
    copy_fn(s * RPT, RPT)

    @pl.when(s == 0)
    def _():
        copy_fn(TAIL_OFF, TAIL)


@functools.partial(
    pl.kernel,
    mesh=_mesh,
    out_type=jax.ShapeDtypeStruct((NC, N, 8), jnp.float32),
    scratch_types=[
        pltpu.VMEM((NCH, K), jnp.int32),
        pltpu.VMEM((K, 8), jnp.float32),
        pltpu.VMEM_SHARED((N, 8), jnp.float32),
    ],
    compiler_params=pltpu.CompilerParams(use_tc_tiling_on_sc=False),
)
def _deg_sc(dst_hbm, ones_hbm, zeros_hbm, out_hbm, idx_v, ones_v, acc_sh):
    c = lax.axis_index("c")
    s = lax.axis_index("s")
    wid = s * NC + c
    pltpu.sync_copy(dst_hbm.at[wid], idx_v)
    pltpu.sync_copy(ones_hbm, ones_v)
    _striped(s, lambda off, n: pltpu.sync_copy(
        zeros_hbm.at[pl.ds(off, n)], acc_sh.at[pl.ds(off, n)]))
    plsc.subcore_barrier()

    def body(i, carry):
        pltpu.sync_copy(ones_v, acc_sh.at[idx_v.at[i]], add=True)
        return carry

    lax.fori_loop(0, NCH, body, 0)
    plsc.subcore_barrier()
    _striped(s, lambda off, n: pltpu.sync_copy(
        acc_sh.at[pl.ds(off, n)], out_hbm.at[c, pl.ds(off, n)]))


@functools.partial(
    pl.kernel,
    mesh=_mesh,
    out_type=jax.ShapeDtypeStruct((NC, N, HID), jnp.float32),
    scratch_types=[
        pltpu.VMEM((NCH, K), jnp.int32),
        pltpu.VMEM((NCH, K), jnp.int32),
        pltpu.VMEM((K, HID), jnp.float32),
        pltpu.VMEM_SHARED((N, HID), jnp.float32),
        pltpu.SemaphoreType.DMA,
    ],
    compiler_params=pltpu.CompilerParams(use_tc_tiling_on_sc=False),
)
def _agg_sc(src_hbm, dst_hbm, y_hbm, zeros_hbm, out_hbm,
            src_v, dst_v, rows_v, acc_sh, sem):
    c = lax.axis_index("c")
    s = lax.axis_index("s")
    wid = s * NC + c
    pltpu.sync_copy(src_hbm.at[wid], src_v)
    pltpu.sync_copy(dst_hbm.at[wid], dst_v)
    _striped(s, lambda off, n: pltpu.sync_copy(
        zeros_hbm.at[pl.ds(off, n)], acc_sh.at[pl.ds(off, n)]))
    plsc.subcore_barrier()

    def body(i, carry):
        pltpu.async_copy(y_hbm.at[src_v.at[i]], rows_v, sem).wait()
        pltpu.sync_copy(rows_v, acc_sh.at[dst_v.at[i]], add=True)
        return carry

    lax.fori_loop(0, NCH, body, 0)
    plsc.subcore_barrier()
    _striped(s, lambda off, n: pltpu.sync_copy(
        acc_sh.at[pl.ds(off, n)], out_hbm.at[c, pl.ds(off, n)]))


R = 1000


def _tcA_body(x_ref, w1_ref, p0_ref, p1_ref, y1_ref, dis_ref):
    deg = 1.0 + p0_ref[:, 0:1] + p1_ref[:, 0:1]
    dis = lax.rsqrt(deg)
    xw = jnp.dot(x_ref[...], w1_ref[...], preferred_element_type=jnp.float32)
    y1_ref[...] = xw * dis
    dis_ref[...] = dis


def _tcA(x, W1, p0, p1):
    return pl.pallas_call(
        _tcA_body,
        grid=(N // R,),
        in_specs=[
            pl.BlockSpec((R, IN_CH), lambda r: (r, 0)),
            pl.BlockSpec((IN_CH, HID), lambda r: (0, 0)),
            pl.BlockSpec((R, 8), lambda r: (r, 0)),
            pl.BlockSpec((R, 8), lambda r: (r, 0)),
        ],
        out_specs=[
            pl.BlockSpec((R, HID), lambda r: (r, 0)),
            pl.BlockSpec((R, 1), lambda r: (r, 0)),
        ],
        out_shape=[
            jax.ShapeDtypeStruct((N, HID), jnp.float32),
            jax.ShapeDtypeStruct((N, 1), jnp.float32),
        ],
    )(x, W1, p0, p1)


def _tcB_body(y1_ref, a0_ref, a1_ref, dis_ref, w2_ref, b1_ref, y2_ref):
    dis = dis_ref[...]
    h = (a0_ref[...] + a1_ref[...] + y1_ref[...]) * dis + b1_ref[...]
    h = jnp.maximum(h, 0.0)
    y2_ref[...] = jnp.dot(h, w2_ref[...],
                          preferred_element_type=jnp.float32) * dis


def _tcB(y1, a0, a1, dis, W2, b1r):
    return pl.pallas_call(
        _tcB_body,
        grid=(N // R,),
        in_specs=[
            pl.BlockSpec((R, HID), lambda r: (r, 0)),
            pl.BlockSpec((R, HID), lambda r: (r, 0)),
            pl.BlockSpec((R, HID), lambda r: (r, 0)),
            pl.BlockSpec((R, 1), lambda r: (r, 0)),
            pl.BlockSpec((HID, HID), lambda r: (0, 0)),
            pl.BlockSpec((1, HID), lambda r: (0, 0)),
        ],
        out_specs=pl.BlockSpec((R, HID), lambda r: (r, 0)),
        out_shape=jax.ShapeDtypeStruct((N, HID), jnp.float32),
    )(y1, a0, a1, dis, W2, b1r)


def _tcC_body(y2_ref, a0_ref, a1_ref, dis_ref, b2_ref, wl_ref, bl_ref, q_ref):
    h = (a0_ref[...] + a1_ref[...] + y2_ref[...]) * dis_ref[...] + b2_ref[...]
    h = jnp.maximum(h, 0.0)
    q_ref[...] = jnp.dot(h, wl_ref[...],
                         preferred_element_type=jnp.float32) + bl_ref[...]


def _tcC(y2, a0, a1, dis, b2r, Wl, blr):
    return pl.pallas_call(
        _tcC_body,
        grid=(N // R,),
        in_specs=[
            pl.BlockSpec((R, HID), lambda r: (r, 0)),
            pl.BlockSpec((R, HID), lambda r: (r, 0)),
            pl.BlockSpec((R, HID), lambda r: (r, 0)),
            pl.BlockSpec((R, 1), lambda r: (r, 0)),
            pl.BlockSpec((1, HID), lambda r: (0, 0)),
            pl.BlockSpec((HID, OUT_CH), lambda r: (0, 0)),
            pl.BlockSpec((1, OUT_CH), lambda r: (0, 0)),
        ],
        out_specs=pl.BlockSpec((R, OUT_CH), lambda r: (r, 0)),
        out_shape=jax.ShapeDtypeStruct((N, OUT_CH), jnp.float32),
    )(y2, a0, a1, dis, b2r, Wl, blr)


def kernel(x, edge_index, W1, b1, W2, b2, Wl, bl):
    src = edge_index[0].reshape(NW, NCH, K)
    dst = edge_index[1].reshape(NW, NCH, K)
    ones8 = jnp.ones((K, 8), jnp.float32)
    z8 = jnp.zeros((N, 8), jnp.float32)
    z64 = jnp.zeros((N, HID), jnp.float32)

    degp = _deg_sc(dst, ones8, z8)
    y1, dis = _tcA(x, W1, degp[0], degp[1])
    agg1 = _agg_sc(src, dst, y1, z64)
    y2 = _tcB(y1, agg1[0], agg1[1], dis, W2, b1.reshape(1, HID))
    agg2 = _agg_sc(src, dst, y2, z64)
    q = _tcC(y2, agg2[0], agg2[1], dis, b2.reshape(1, HID),
             Wl, bl.reshape(1, OUT_CH))
    return q


# speedup vs baseline: 26.7572x; 1.1586x over previous
import functools

import jax
import jax.numpy as jnp
from jax import lax
from jax.experimental import pallas as pl
from jax.experimental.pallas import tpu as pltpu
from jax.experimental.pallas import tpu_sc as plsc

N = 10000
E = 320000
IN_CH = 128
HID = 64
OUT_CH = 2

NC = 2
NS = 16
NW = NC * NS
EPT = E // NW
K = 80
NCH = EPT // K
RPT = 624
TAIL = N - NS * RPT
TAIL_OFF = NS * RPT

_mesh = plsc.VectorSubcoreMesh(core_axis_name="c", subcore_axis_name="s")


def _striped(s, copy_fn):
    copy_fn(s * RPT, RPT)

    @pl.when(s == 0)
    def _():
        copy_fn(TAIL_OFF, TAIL)


@functools.partial(
    pl.kernel,
    mesh=_mesh,
    out_type=jax.ShapeDtypeStruct((NC, N, 8), jnp.float32),
    scratch_types=[
        pltpu.VMEM((NCH, K), jnp.int32),
        pltpu.VMEM((K, 8), jnp.float32),
        pltpu.VMEM_SHARED((N, 8), jnp.float32),
    ],
    compiler_params=pltpu.CompilerParams(use_tc_tiling_on_sc=False),
)
def _deg_sc(dst_hbm, ones_hbm, zeros_hbm, out_hbm, idx_v, ones_v, acc_sh):
    c = lax.axis_index("c")
    s = lax.axis_index("s")
    wid = s * NC + c
    pltpu.sync_copy(dst_hbm.at[wid], idx_v)
    pltpu.sync_copy(ones_hbm, ones_v)
    _striped(s, lambda off, n: pltpu.sync_copy(
        zeros_hbm.at[pl.ds(off, n)], acc_sh.at[pl.ds(off, n)]))
    plsc.subcore_barrier()

    def body(i, carry):
        pltpu.sync_copy(ones_v, acc_sh.at[idx_v.at[i]], add=True)
        return carry

    lax.fori_loop(0, NCH, body, 0)
    plsc.subcore_barrier()
    _striped(s, lambda off, n: pltpu.sync_copy(
        acc_sh.at[pl.ds(off, n)], out_hbm.at[c, pl.ds(off, n)]))


@functools.partial(
    pl.kernel,
    mesh=_mesh,
    out_type=jax.ShapeDtypeStruct((NC, N, HID), jnp.float32),
    scratch_types=[
        pltpu.VMEM((NCH, K), jnp.int32),
        pltpu.VMEM((NCH, K), jnp.int32),
        pltpu.VMEM((K, HID), jnp.float32),
        pltpu.VMEM((K, HID), jnp.float32),
        pltpu.SemaphoreType.DMA,
        pltpu.SemaphoreType.DMA,
        pltpu.VMEM_SHARED((N, HID), jnp.float32),
    ],
    compiler_params=pltpu.CompilerParams(use_tc_tiling_on_sc=False),
)
def _agg_sc(src_hbm, dst_hbm, y_hbm, zeros_hbm, out_hbm,
            src_v, dst_v, rows0, rows1, sem0, sem1, acc_sh):
    c = lax.axis_index("c")
    s = lax.axis_index("s")
    wid = s * NC + c
    pltpu.sync_copy(src_hbm.at[wid], src_v)
    pltpu.sync_copy(dst_hbm.at[wid], dst_v)
    _striped(s, lambda off, n: pltpu.sync_copy(
        zeros_hbm.at[pl.ds(off, n)], acc_sh.at[pl.ds(off, n)]))
    plsc.subcore_barrier()

    bufs = (rows0, rows1)
    sems = (sem0, sem1)
    pltpu.async_copy(y_hbm.at[src_v.at[0]], rows0, sem0)

    def _step(i, b):
        pltpu.make_async_copy(y_hbm.at[src_v.at[i]], bufs[b], sems[b]).wait()
        pltpu.async_copy(y_hbm.at[src_v.at[i + 1]], bufs[1 - b], sems[1 - b])
        pltpu.sync_copy(bufs[b], acc_sh.at[dst_v.at[i]], add=True)

    def body(j, carry):
        _step(2 * j, 0)
        _step(2 * j + 1, 1)
        return carry

    lax.fori_loop(0, (NCH - 1) // 2, body, 0)
    i_last = NCH - 1
    pltpu.make_async_copy(y_hbm.at[src_v.at[i_last]], rows0, sem0).wait()
    pltpu.sync_copy(rows0, acc_sh.at[dst_v.at[i_last]], add=True)

    plsc.subcore_barrier()
    _striped(s, lambda off, n: pltpu.sync_copy(
        acc_sh.at[pl.ds(off, n)], out_hbm.at[c, pl.ds(off, n)]))


R = 1000


def _tcA_body(x_ref, w1_ref, p0_ref, p1_ref, y1_ref, dis_ref):
    deg = 1.0 + p0_ref[:, 0:1] + p1_ref[:, 0:1]
    dis = lax.rsqrt(deg)
    xw = jnp.dot(x_ref[...], w1_ref[...], preferred_element_type=jnp.float32)
    y1_ref[...] = xw * dis
    dis_ref[...] = dis


def _tcA(x, W1, p0, p1):
    return pl.pallas_call(
        _tcA_body,
        grid=(N // R,),
        in_specs=[
            pl.BlockSpec((R, IN_CH), lambda r: (r, 0)),
            pl.BlockSpec((IN_CH, HID), lambda r: (0, 0)),
            pl.BlockSpec((R, 8), lambda r: (r, 0)),
            pl.BlockSpec((R, 8), lambda r: (r, 0)),
        ],
        out_specs=[
            pl.BlockSpec((R, HID), lambda r: (r, 0)),
            pl.BlockSpec((R, 1), lambda r: (r, 0)),
        ],
        out_shape=[
            jax.ShapeDtypeStruct((N, HID), jnp.float32),
            jax.ShapeDtypeStruct((N, 1), jnp.float32),
        ],
    )(x, W1, p0, p1)


def _tcB_body(y1_ref, a0_ref, a1_ref, dis_ref, w2_ref, b1_ref, y2_ref):
    dis = dis_ref[...]
    h = (a0_ref[...] + a1_ref[...] + y1_ref[...]) * dis + b1_ref[...]
    h = jnp.maximum(h, 0.0)
    y2_ref[...] = jnp.dot(h, w2_ref[...],
                          preferred_element_type=jnp.float32) * dis


def _tcB(y1, a0, a1, dis, W2, b1r):
    return pl.pallas_call(
        _tcB_body,
        grid=(N // R,),
        in_specs=[
            pl.BlockSpec((R, HID), lambda r: (r, 0)),
            pl.BlockSpec((R, HID), lambda r: (r, 0)),
            pl.BlockSpec((R, HID), lambda r: (r, 0)),
            pl.BlockSpec((R, 1), lambda r: (r, 0)),
            pl.BlockSpec((HID, HID), lambda r: (0, 0)),
            pl.BlockSpec((1, HID), lambda r: (0, 0)),
        ],
        out_specs=pl.BlockSpec((R, HID), lambda r: (r, 0)),
        out_shape=jax.ShapeDtypeStruct((N, HID), jnp.float32),
    )(y1, a0, a1, dis, W2, b1r)


def _tcC_body(y2_ref, a0_ref, a1_ref, dis_ref, b2_ref, wl_ref, bl_ref, q_ref):
    h = (a0_ref[...] + a1_ref[...] + y2_ref[...]) * dis_ref[...] + b2_ref[...]
    h = jnp.maximum(h, 0.0)
    q_ref[...] = jnp.dot(h, wl_ref[...],
                         preferred_element_type=jnp.float32) + bl_ref[...]


def _tcC(y2, a0, a1, dis, b2r, Wl, blr):
    return pl.pallas_call(
        _tcC_body,
        grid=(N // R,),
        in_specs=[
            pl.BlockSpec((R, HID), lambda r: (r, 0)),
            pl.BlockSpec((R, HID), lambda r: (r, 0)),
            pl.BlockSpec((R, HID), lambda r: (r, 0)),
            pl.BlockSpec((R, 1), lambda r: (r, 0)),
            pl.BlockSpec((1, HID), lambda r: (0, 0)),
            pl.BlockSpec((HID, OUT_CH), lambda r: (0, 0)),
            pl.BlockSpec((1, OUT_CH), lambda r: (0, 0)),
        ],
        out_specs=pl.BlockSpec((R, OUT_CH), lambda r: (r, 0)),
        out_shape=jax.ShapeDtypeStruct((N, OUT_CH), jnp.float32),
    )(y2, a0, a1, dis, b2r, Wl, blr)


def kernel(x, edge_index, W1, b1, W2, b2, Wl, bl):
    src = edge_index[0].reshape(NW, NCH, K)
    dst = edge_index[1].reshape(NW, NCH, K)
    ones8 = jnp.ones((K, 8), jnp.float32)
    z8 = jnp.zeros((N, 8), jnp.float32)
    z64 = jnp.zeros((N, HID), jnp.float32)

    degp = _deg_sc(dst, ones8, z8)
    y1, dis = _tcA(x, W1, degp[0], degp[1])
    agg1 = _agg_sc(src, dst, y1, z64)
    y2 = _tcB(y1, agg1[0], agg1[1], dis, W2, b1.reshape(1, HID))
    agg2 = _agg_sc(src, dst, y2, z64)
    q = _tcC(y2, agg2[0], agg2[1], dis, b2.reshape(1, HID),
             Wl, bl.reshape(1, OUT_CH))
    return q


# speedup vs baseline: 34.3423x; 1.2835x over previous
import functools

import jax
import jax.numpy as jnp
from jax import lax
from jax.experimental import pallas as pl
from jax.experimental.pallas import tpu as pltpu
from jax.experimental.pallas import tpu_sc as plsc

N = 10000
E = 320000
IN_CH = 128
HID = 64
OUT_CH = 2

NC = 2
NS = 16
NW = NC * NS
EPT = E // NW
K = 80
NCH = EPT // K
RPT = 624
TAIL = N - NS * RPT
TAIL_OFF = NS * RPT

_mesh = plsc.VectorSubcoreMesh(core_axis_name="c", subcore_axis_name="s")


def _striped(s, copy_fn):
    copy_fn(s * RPT, RPT)

    @pl.when(s == 0)
    def _():
        copy_fn(TAIL_OFF, TAIL)


@functools.partial(
    pl.kernel,
    mesh=_mesh,
    out_type=jax.ShapeDtypeStruct((NC, N, 8), jnp.float32),
    scratch_types=[
        pltpu.VMEM((NCH, K), jnp.int32),
        pltpu.VMEM((K, 8), jnp.float32),
        pltpu.VMEM_SHARED((N, 8), jnp.float32),
    ],
    compiler_params=pltpu.CompilerParams(use_tc_tiling_on_sc=False),
)
def _deg_sc(dst_hbm, ones_hbm, zeros_hbm, out_hbm, idx_v, ones_v, acc_sh):
    c = lax.axis_index("c")
    s = lax.axis_index("s")
    wid = s * NC + c
    pltpu.sync_copy(dst_hbm.at[wid], idx_v)
    pltpu.sync_copy(ones_hbm, ones_v)
    _striped(s, lambda off, n: pltpu.sync_copy(
        zeros_hbm.at[pl.ds(off, n)], acc_sh.at[pl.ds(off, n)]))
    plsc.subcore_barrier()

    def body(i, carry):
        pltpu.sync_copy(ones_v, acc_sh.at[idx_v.at[i]], add=True)
        return carry

    lax.fori_loop(0, NCH, body, 0)
    plsc.subcore_barrier()
    _striped(s, lambda off, n: pltpu.sync_copy(
        acc_sh.at[pl.ds(off, n)], out_hbm.at[c, pl.ds(off, n)]))


@functools.partial(
    pl.kernel,
    mesh=_mesh,
    out_type=jax.ShapeDtypeStruct((NC, N, HID), jnp.float32),
    scratch_types=[
        pltpu.VMEM((NCH, K), jnp.int32),
        pltpu.VMEM((NCH, K), jnp.int32),
        [pltpu.VMEM((K, HID), jnp.float32)] * 4,
        [pltpu.SemaphoreType.DMA] * 4,
        [pltpu.SemaphoreType.DMA] * 4,
        pltpu.VMEM_SHARED((N, HID), jnp.float32),
    ],
    compiler_params=pltpu.CompilerParams(use_tc_tiling_on_sc=False),
)
def _agg_sc(src_hbm, dst_hbm, y_hbm, zeros_hbm, out_hbm,
            src_v, dst_v, bufs, gsems, ssems, acc_sh):
    c = lax.axis_index("c")
    s = lax.axis_index("s")
    wid = s * NC + c
    pltpu.sync_copy(src_hbm.at[wid], src_v)
    pltpu.sync_copy(dst_hbm.at[wid], dst_v)
    _striped(s, lambda off, n: pltpu.sync_copy(
        zeros_hbm.at[pl.ds(off, n)], acc_sh.at[pl.ds(off, n)]))
    plsc.subcore_barrier()

    def _gather(i, b):
        pltpu.async_copy(y_hbm.at[src_v.at[i]], bufs[b], gsems[b])

    def _wait_gather(i, b):
        pltpu.make_async_copy(y_hbm.at[src_v.at[i]], bufs[b], gsems[b]).wait()

    def _scatter(i, b):
        pltpu.async_copy(bufs[b], acc_sh.at[dst_v.at[i]], ssems[b], add=True)

    def _wait_scatter(i, b):
        pltpu.make_async_copy(bufs[b], acc_sh.at[dst_v.at[i]],
                              ssems[b]).wait()

    def _step(i, b, wait_prev_scatter):
        _wait_gather(i, b)
        _scatter(i, b)
        b2 = (b + 2) % 4
        if wait_prev_scatter:
            _wait_scatter(i - 2, b2)
        _gather(i + 2, b2)

    _gather(0, 0)
    _gather(1, 1)
    _step(0, 0, False)
    _step(1, 1, False)
    _step(2, 2, True)
    _step(3, 3, True)

    def body(j, carry):
        i = 4 * j
        _step(i, 0, True)
        _step(i + 1, 1, True)
        _step(i + 2, 2, True)
        _step(i + 3, 3, True)
        return carry

    lax.fori_loop(1, 30, body, 0)

    _wait_gather(120, 0)
    _scatter(120, 0)
    _wait_scatter(118, 2)
    _gather(122, 2)
    _wait_gather(121, 1)
    _scatter(121, 1)
    _wait_scatter(119, 3)
    _gather(123, 3)
    _wait_gather(122, 2)
    _scatter(122, 2)
    _wait_scatter(120, 0)
    _gather(124, 0)
    _wait_gather(123, 3)
    _scatter(123, 3)
    _wait_gather(124, 0)
    _scatter(124, 0)
    _wait_scatter(121, 1)
    _wait_scatter(122, 2)
    _wait_scatter(123, 3)
    _wait_scatter(124, 0)

    plsc.subcore_barrier()
    _striped(s, lambda off, n: pltpu.sync_copy(
        acc_sh.at[pl.ds(off, n)], out_hbm.at[c, pl.ds(off, n)]))


R = 1000


def _tcA1_body(x_ref, w1_ref, xw_ref):
    xw_ref[...] = jnp.dot(x_ref[...], w1_ref[...],
                          preferred_element_type=jnp.float32)


def _tcA1(x, W1):
    return pl.pallas_call(
        _tcA1_body,
        grid=(N // R,),
        in_specs=[
            pl.BlockSpec((R, IN_CH), lambda r: (r, 0)),
            pl.BlockSpec((IN_CH, HID), lambda r: (0, 0)),
        ],
        out_specs=pl.BlockSpec((R, HID), lambda r: (r, 0)),
        out_shape=jax.ShapeDtypeStruct((N, HID), jnp.float32),
    )(x, W1)


def _tcA2_body(xw_ref, p0_ref, p1_ref, y1_ref, dis_ref):
    deg = 1.0 + p0_ref[:, 0:1] + p1_ref[:, 0:1]
    dis = lax.rsqrt(deg)
    y1_ref[...] = xw_ref[...] * dis
    dis_ref[...] = dis


def _tcA2(xw, p0, p1):
    return pl.pallas_call(
        _tcA2_body,
        grid=(N // R,),
        in_specs=[
            pl.BlockSpec((R, HID), lambda r: (r, 0)),
            pl.BlockSpec((R, 8), lambda r: (r, 0)),
            pl.BlockSpec((R, 8), lambda r: (r, 0)),
        ],
        out_specs=[
            pl.BlockSpec((R, HID), lambda r: (r, 0)),
            pl.BlockSpec((R, 1), lambda r: (r, 0)),
        ],
        out_shape=[
            jax.ShapeDtypeStruct((N, HID), jnp.float32),
            jax.ShapeDtypeStruct((N, 1), jnp.float32),
        ],
    )(xw, p0, p1)


def _tcB_body(y1_ref, a0_ref, a1_ref, dis_ref, w2_ref, b1_ref, y2_ref):
    dis = dis_ref[...]
    h = (a0_ref[...] + a1_ref[...] + y1_ref[...]) * dis + b1_ref[...]
    h = jnp.maximum(h, 0.0)
    y2_ref[...] = jnp.dot(h, w2_ref[...],
                          preferred_element_type=jnp.float32) * dis


def _tcB(y1, a0, a1, dis, W2, b1r):
    return pl.pallas_call(
        _tcB_body,
        grid=(N // R,),
        in_specs=[
            pl.BlockSpec((R, HID), lambda r: (r, 0)),
            pl.BlockSpec((R, HID), lambda r: (r, 0)),
            pl.BlockSpec((R, HID), lambda r: (r, 0)),
            pl.BlockSpec((R, 1), lambda r: (r, 0)),
            pl.BlockSpec((HID, HID), lambda r: (0, 0)),
            pl.BlockSpec((1, HID), lambda r: (0, 0)),
        ],
        out_specs=pl.BlockSpec((R, HID), lambda r: (r, 0)),
        out_shape=jax.ShapeDtypeStruct((N, HID), jnp.float32),
    )(y1, a0, a1, dis, W2, b1r)


def _tcC_body(y2_ref, a0_ref, a1_ref, dis_ref, b2_ref, wl_ref, bl_ref, q_ref):
    h = (a0_ref[...] + a1_ref[...] + y2_ref[...]) * dis_ref[...] + b2_ref[...]
    h = jnp.maximum(h, 0.0)
    q_ref[...] = jnp.dot(h, wl_ref[...],
                         preferred_element_type=jnp.float32) + bl_ref[...]


def _tcC(y2, a0, a1, dis, b2r, Wl, blr):
    return pl.pallas_call(
        _tcC_body,
        grid=(N // R,),
        in_specs=[
            pl.BlockSpec((R, HID), lambda r: (r, 0)),
            pl.BlockSpec((R, HID), lambda r: (r, 0)),
            pl.BlockSpec((R, HID), lambda r: (r, 0)),
            pl.BlockSpec((R, 1), lambda r: (r, 0)),
            pl.BlockSpec((1, HID), lambda r: (0, 0)),
            pl.BlockSpec((HID, OUT_CH), lambda r: (0, 0)),
            pl.BlockSpec((1, OUT_CH), lambda r: (0, 0)),
        ],
        out_specs=pl.BlockSpec((R, OUT_CH), lambda r: (r, 0)),
        out_shape=jax.ShapeDtypeStruct((N, OUT_CH), jnp.float32),
    )(y2, a0, a1, dis, b2r, Wl, blr)


def kernel(x, edge_index, W1, b1, W2, b2, Wl, bl):
    src = edge_index[0].reshape(NW, NCH, K)
    dst = edge_index[1].reshape(NW, NCH, K)
    ones8 = jnp.ones((K, 8), jnp.float32)
    z8 = jnp.zeros((N, 8), jnp.float32)
    z64 = jnp.zeros((N, HID), jnp.float32)

    degp = _deg_sc(dst, ones8, z8)
    xw = _tcA1(x, W1)
    y1, dis = _tcA2(xw, degp[0], degp[1])
    agg1 = _agg_sc(src, dst, y1, z64)
    y2 = _tcB(y1, agg1[0], agg1[1], dis, W2, b1.reshape(1, HID))
    agg2 = _agg_sc(src, dst, y2, z64)
    q = _tcC(y2, agg2[0], agg2[1], dis, b2.reshape(1, HID),
             Wl, bl.reshape(1, OUT_CH))
    return q


# speedup vs baseline: 37.9931x; 1.1063x over previous
import functools

import jax
import jax.numpy as jnp
from jax import lax
from jax.experimental import pallas as pl
from jax.experimental.pallas import tpu as pltpu
from jax.experimental.pallas import tpu_sc as plsc

N = 10000
E = 320000
IN_CH = 128
HID = 64
OUT_CH = 2

NC = 2
NS = 16
NW = NC * NS
EPT = E // NW
K = 80
NCH = EPT // K
RPT = 624
TAIL = N - NS * RPT
TAIL_OFF = NS * RPT

_mesh = plsc.VectorSubcoreMesh(core_axis_name="c", subcore_axis_name="s")


def _striped(s, copy_fn):
    copy_fn(s * RPT, RPT)

    @pl.when(s == 0)
    def _():
        copy_fn(TAIL_OFF, TAIL)


@functools.partial(
    pl.kernel,
    mesh=_mesh,
    out_type=jax.ShapeDtypeStruct((NC, N, 8), jnp.float32),
    scratch_types=[
        pltpu.VMEM((NCH, K), jnp.int32),
        pltpu.VMEM((K, 8), jnp.float32),
        [pltpu.SemaphoreType.DMA] * 5,
        pltpu.VMEM_SHARED((N, 8), jnp.float32),
    ],
    compiler_params=pltpu.CompilerParams(use_tc_tiling_on_sc=False),
)
def _deg_sc(dst_hbm, ones_hbm, zeros_hbm, out_hbm, idx_v, ones_v, sems,
            acc_sh):
    c = lax.axis_index("c")
    s = lax.axis_index("s")
    wid = s * NC + c
    pltpu.sync_copy(dst_hbm.at[wid], idx_v)
    pltpu.sync_copy(ones_hbm, ones_v)
    _striped(s, lambda off, n: pltpu.sync_copy(
        zeros_hbm.at[pl.ds(off, n)], acc_sh.at[pl.ds(off, n)]))
    plsc.subcore_barrier()

    def _scat(i, b):
        pltpu.async_copy(ones_v, acc_sh.at[idx_v.at[i]], sems[b], add=True)

    def _wait(i, b):
        pltpu.make_async_copy(ones_v, acc_sh.at[idx_v.at[i]], sems[b]).wait()

    for b in range(5):
        _scat(b, b)

    def body(j, carry):
        i = 5 * j
        for b in range(5):
            _wait(i - 5 + b, b)
            _scat(i + b, b)
        return carry

    lax.fori_loop(1, NCH // 5, body, 0)
    for b in range(5):
        _wait(120 + b, b)
    plsc.subcore_barrier()
    _striped(s, lambda off, n: pltpu.sync_copy(
        acc_sh.at[pl.ds(off, n)], out_hbm.at[c, pl.ds(off, n)]))


@functools.partial(
    pl.kernel,
    mesh=_mesh,
    out_type=jax.ShapeDtypeStruct((NC, N, HID), jnp.float32),
    scratch_types=[
        pltpu.VMEM((NCH, K), jnp.int32),
        pltpu.VMEM((NCH, K), jnp.int32),
        [pltpu.VMEM((K, HID), jnp.float32)] * 5,
        [pltpu.SemaphoreType.DMA] * 5,
        [pltpu.SemaphoreType.DMA] * 5,
        pltpu.VMEM_SHARED((N, HID), jnp.float32),
    ],
    compiler_params=pltpu.CompilerParams(use_tc_tiling_on_sc=False),
)
def _agg_sc(src_hbm, dst_hbm, y_hbm, zeros_hbm, out_hbm,
            src_v, dst_v, bufs, gsems, ssems, acc_sh):
    c = lax.axis_index("c")
    s = lax.axis_index("s")
    wid = s * NC + c
    pltpu.sync_copy(src_hbm.at[wid], src_v)
    pltpu.sync_copy(dst_hbm.at[wid], dst_v)
    _striped(s, lambda off, n: pltpu.sync_copy(
        zeros_hbm.at[pl.ds(off, n)], acc_sh.at[pl.ds(off, n)]))
    plsc.subcore_barrier()

    RING = 5

    def _gather(i, b):
        pltpu.async_copy(y_hbm.at[src_v.at[i]], bufs[b], gsems[b])

    def _wait_gather(i, b):
        pltpu.make_async_copy(y_hbm.at[src_v.at[i]], bufs[b], gsems[b]).wait()

    def _scatter(i, b):
        pltpu.async_copy(bufs[b], acc_sh.at[dst_v.at[i]], ssems[b], add=True)

    def _wait_scatter(i, b):
        pltpu.make_async_copy(bufs[b], acc_sh.at[dst_v.at[i]],
                              ssems[b]).wait()

    def _step(i, b, guards):
        _wait_gather(i, b)
        _scatter(i, b)
        b2 = (b + 2) % RING
        if guards:
            def _pf():
                _wait_scatter(i - 3, b2)
                _gather(i + 2, b2)
            if guards == "tail":
                pl.when(i + 2 < NCH)(_pf)
            else:
                _pf()
        else:
            _gather(i + 2, b2)

    _gather(0, 0)
    _gather(1, 1)
    _step(0, 0, None)
    _step(1, 1, None)
    _step(2, 2, None)
    _step(3, 3, "steady")
    _step(4, 4, "steady")

    def body(j, carry):
        i = RING * j
        _step(i, 0, "steady")
        _step(i + 1, 1, "steady")
        _step(i + 2, 2, "steady")
        _step(i + 3, 3, "tail")
        _step(i + 4, 4, "tail")
        return carry

    lax.fori_loop(1, NCH // RING, body, 0)

    _wait_scatter(120, 0)
    _wait_scatter(121, 1)
    _wait_scatter(122, 2)
    _wait_scatter(123, 3)
    _wait_scatter(124, 4)

    plsc.subcore_barrier()
    _striped(s, lambda off, n: pltpu.sync_copy(
        acc_sh.at[pl.ds(off, n)], out_hbm.at[c, pl.ds(off, n)]))


R = 1000


def _tcA1_body(x_ref, w1_ref, xw_ref):
    xw_ref[...] = jnp.dot(x_ref[...], w1_ref[...],
                          preferred_element_type=jnp.float32)


def _tcA1(x, W1):
    return pl.pallas_call(
        _tcA1_body,
        grid=(N // R,),
        in_specs=[
            pl.BlockSpec((R, IN_CH), lambda r: (r, 0)),
            pl.BlockSpec((IN_CH, HID), lambda r: (0, 0)),
        ],
        out_specs=pl.BlockSpec((R, HID), lambda r: (r, 0)),
        out_shape=jax.ShapeDtypeStruct((N, HID), jnp.float32),
    )(x, W1)


def _tcA2_body(xw_ref, p_ref, y1_ref, dis_ref):
    deg = 1.0 + p_ref[0, :, 0:1] + p_ref[1, :, 0:1]
    dis = lax.rsqrt(deg)
    y1_ref[...] = xw_ref[...] * dis
    dis_ref[...] = dis


def _tcA2(xw, degp):
    return pl.pallas_call(
        _tcA2_body,
        grid=(N // R,),
        in_specs=[
            pl.BlockSpec((R, HID), lambda r: (r, 0)),
            pl.BlockSpec((2, R, 8), lambda r: (0, r, 0)),
        ],
        out_specs=[
            pl.BlockSpec((R, HID), lambda r: (r, 0)),
            pl.BlockSpec((R, 1), lambda r: (r, 0)),
        ],
        out_shape=[
            jax.ShapeDtypeStruct((N, HID), jnp.float32),
            jax.ShapeDtypeStruct((N, 1), jnp.float32),
        ],
    )(xw, degp)


def _tcB_body(y1_ref, a_ref, dis_ref, w2_ref, b1_ref, y2_ref):
    dis = dis_ref[...]
    h = (a_ref[0] + a_ref[1] + y1_ref[...]) * dis + b1_ref[...]
    h = jnp.maximum(h, 0.0)
    y2_ref[...] = jnp.dot(h, w2_ref[...],
                          preferred_element_type=jnp.float32) * dis


def _tcB(y1, agg, dis, W2, b1r):
    return pl.pallas_call(
        _tcB_body,
        grid=(N // R,),
        in_specs=[
            pl.BlockSpec((R, HID), lambda r: (r, 0)),
            pl.BlockSpec((2, R, HID), lambda r: (0, r, 0)),
            pl.BlockSpec((R, 1), lambda r: (r, 0)),
            pl.BlockSpec((HID, HID), lambda r: (0, 0)),
            pl.BlockSpec((1, HID), lambda r: (0, 0)),
        ],
        out_specs=pl.BlockSpec((R, HID), lambda r: (r, 0)),
        out_shape=jax.ShapeDtypeStruct((N, HID), jnp.float32),
    )(y1, agg, dis, W2, b1r)


def _tcC_body(y2_ref, a_ref, dis_ref, b2_ref, wl_ref, bl_ref, q_ref):
    h = (a_ref[0] + a_ref[1] + y2_ref[...]) * dis_ref[...] + b2_ref[...]
    h = jnp.maximum(h, 0.0)
    q_ref[...] = jnp.dot(h, wl_ref[...],
                         preferred_element_type=jnp.float32) + bl_ref[...]


def _tcC(y2, agg, dis, b2r, Wl, blr):
    return pl.pallas_call(
        _tcC_body,
        grid=(N // R,),
        in_specs=[
            pl.BlockSpec((R, HID), lambda r: (r, 0)),
            pl.BlockSpec((2, R, HID), lambda r: (0, r, 0)),
            pl.BlockSpec((R, 1), lambda r: (r, 0)),
            pl.BlockSpec((1, HID), lambda r: (0, 0)),
            pl.BlockSpec((HID, OUT_CH), lambda r: (0, 0)),
            pl.BlockSpec((1, OUT_CH), lambda r: (0, 0)),
        ],
        out_specs=pl.BlockSpec((R, OUT_CH), lambda r: (r, 0)),
        out_shape=jax.ShapeDtypeStruct((N, OUT_CH), jnp.float32),
    )(y2, agg, dis, b2r, Wl, blr)


def kernel(x, edge_index, W1, b1, W2, b2, Wl, bl):
    src = edge_index[0].reshape(NW, NCH, K)
    dst = edge_index[1].reshape(NW, NCH, K)
    ones8 = jnp.ones((K, 8), jnp.float32)
    z8 = jnp.zeros((N, 8), jnp.float32)
    z64 = jnp.zeros((N, HID), jnp.float32)

    degp = _deg_sc(dst, ones8, z8)
    xw = _tcA1(x, W1)
    y1, dis = _tcA2(xw, degp)
    agg1 = _agg_sc(src, dst, y1, z64)
    y2 = _tcB(y1, agg1, dis, W2, b1.reshape(1, HID))
    agg2 = _agg_sc(src, dst, y2, z64)
    q = _tcC(y2, agg2, dis, b2.reshape(1, HID),
             Wl, bl.reshape(1, OUT_CH))
    return q


# speedup vs baseline: 44.9827x; 1.1840x over previous
import functools

import jax
import jax.numpy as jnp
import numpy as np
from jax import lax
from jax.experimental import pallas as pl
from jax.experimental.pallas import tpu as pltpu
from jax.experimental.pallas import tpu_sc as plsc

N = 10000
E = 320000
IN_CH = 128
HID = 64
OUT_CH = 2

NC = 2
NS = 16
NW = NC * NS
EPT = E // NW
K = 80
NCH = EPT // K
RPT = 624
TAIL = N - NS * RPT
TAIL_OFF = NS * RPT

NV = N * HID // 128
DV = N * 8 // 128

_mesh = plsc.VectorSubcoreMesh(core_axis_name="c", subcore_axis_name="s")


def _striped(s, copy_fn):
    copy_fn(s * RPT, RPT)

    @pl.when(s == 0)
    def _():
        copy_fn(TAIL_OFF, TAIL)


@functools.partial(
    pl.kernel,
    mesh=_mesh,
    out_type=[
        jax.ShapeDtypeStruct((N, 8), jnp.float32),
        jax.ShapeDtypeStruct((N, 8), jnp.float32),
    ],
    scratch_types=[
        pltpu.VMEM((NCH, K), jnp.int32),
        pltpu.VMEM((K, 8), jnp.float32),
        [pltpu.SemaphoreType.DMA] * 5,
        pltpu.VMEM_SHARED((N, 8), jnp.float32),
    ],
    compiler_params=pltpu.CompilerParams(use_tc_tiling_on_sc=False),
)
def _deg_sc(dst_hbm, ones_hbm, zeros_hbm, out0_hbm, out1_hbm, idx_v, ones_v,
            sems, acc_sh):
    c = lax.axis_index("c")
    s = lax.axis_index("s")
    wid = s * NC + c
    pltpu.sync_copy(dst_hbm.at[wid], idx_v)
    pltpu.sync_copy(ones_hbm, ones_v)
    _striped(s, lambda off, n: pltpu.sync_copy(
        zeros_hbm.at[pl.ds(off, n)], acc_sh.at[pl.ds(off, n)]))
    plsc.subcore_barrier()

    def _scat(i, b):
        pltpu.async_copy(ones_v, acc_sh.at[idx_v.at[i]], sems[b], add=True)

    def _wait(i, b):
        pltpu.make_async_copy(ones_v, acc_sh.at[idx_v.at[i]], sems[b]).wait()

    for b in range(5):
        _scat(b, b)

    def body(j, carry):
        i = 5 * j
        for b in range(5):
            _wait(i - 5 + b, b)
            _scat(i + b, b)
        return carry

    lax.fori_loop(1, NCH // 5, body, 0)
    for b in range(5):
        _wait(120 + b, b)
    plsc.subcore_barrier()

    @pl.when(c == 0)
    def _():
        _striped(s, lambda off, n: pltpu.sync_copy(
            acc_sh.at[pl.ds(off, n)], out0_hbm.at[pl.ds(off, n)]))

    @pl.when(c == 1)
    def _():
        _striped(s, lambda off, n: pltpu.sync_copy(
            acc_sh.at[pl.ds(off, n)], out1_hbm.at[pl.ds(off, n)]))


@functools.partial(
    pl.kernel,
    mesh=_mesh,
    out_type=[
        jax.ShapeDtypeStruct((N, HID), jnp.float32),
        jax.ShapeDtypeStruct((N, HID), jnp.float32),
    ],
    scratch_types=[
        pltpu.VMEM((NCH, K), jnp.int32),
        pltpu.VMEM((NCH, K), jnp.int32),
        [pltpu.VMEM((K, HID), jnp.float32)] * 5,
        [pltpu.SemaphoreType.DMA] * 5,
        [pltpu.SemaphoreType.DMA] * 5,
        pltpu.VMEM_SHARED((N, HID), jnp.float32),
    ],
    compiler_params=pltpu.CompilerParams(use_tc_tiling_on_sc=False),
)
def _agg_sc(src_hbm, dst_hbm, y_hbm, zeros_hbm, out0_hbm, out1_hbm,
            src_v, dst_v, bufs, gsems, ssems, acc_sh):
    c = lax.axis_index("c")
    s = lax.axis_index("s")
    wid = s * NC + c
    pltpu.sync_copy(src_hbm.at[wid], src_v)
    pltpu.sync_copy(dst_hbm.at[wid], dst_v)
    _striped(s, lambda off, n: pltpu.sync_copy(
        zeros_hbm.at[pl.ds(off, n)], acc_sh.at[pl.ds(off, n)]))
    plsc.subcore_barrier()

    RING = 5

    def _gather(i, b):
        pltpu.async_copy(y_hbm.at[src_v.at[i]], bufs[b], gsems[b])

    def _wait_gather(i, b):
        pltpu.make_async_copy(y_hbm.at[src_v.at[i]], bufs[b], gsems[b]).wait()

    def _scatter(i, b):
        pltpu.async_copy(bufs[b], acc_sh.at[dst_v.at[i]], ssems[b], add=True)

    def _wait_scatter(i, b):
        pltpu.make_async_copy(bufs[b], acc_sh.at[dst_v.at[i]],
                              ssems[b]).wait()

    def _step(i, b, guards):
        _wait_gather(i, b)
        _scatter(i, b)
        b2 = (b + 2) % RING
        if guards:
            def _pf():
                _wait_scatter(i - 3, b2)
                _gather(i + 2, b2)
            if guards == "tail":
                pl.when(i + 2 < NCH)(_pf)
            else:
                _pf()
        else:
            _gather(i + 2, b2)

    _gather(0, 0)
    _gather(1, 1)
    _step(0, 0, None)
    _step(1, 1, None)
    _step(2, 2, None)
    _step(3, 3, "steady")
    _step(4, 4, "steady")

    def body(j, carry):
        i = RING * j
        _step(i, 0, "steady")
        _step(i + 1, 1, "steady")
        _step(i + 2, 2, "steady")
        _step(i + 3, 3, "tail")
        _step(i + 4, 4, "tail")
        return carry

    lax.fori_loop(1, NCH // RING, body, 0)

    _wait_scatter(120, 0)
    _wait_scatter(121, 1)
    _wait_scatter(122, 2)
    _wait_scatter(123, 3)
    _wait_scatter(124, 4)

    plsc.subcore_barrier()

    @pl.when(c == 0)
    def _():
        _striped(s, lambda off, n: pltpu.sync_copy(
            acc_sh.at[pl.ds(off, n)], out0_hbm.at[pl.ds(off, n)]))

    @pl.when(c == 1)
    def _():
        _striped(s, lambda off, n: pltpu.sync_copy(
            acc_sh.at[pl.ds(off, n)], out1_hbm.at[pl.ds(off, n)]))


RV = 1000
GRID = NV // RV


def _tcA1_body(x_ref, w1_ref, xw_ref):
    xw_ref[...] = jnp.dot(x_ref[...], w1_ref[...],
                          preferred_element_type=jnp.float32)


def _tcA1(xv, W1v):
    return pl.pallas_call(
        _tcA1_body,
        grid=(GRID,),
        in_specs=[
            pl.BlockSpec((RV, 2 * IN_CH), lambda r: (r, 0)),
            pl.BlockSpec((2 * IN_CH, 128), lambda r: (0, 0)),
        ],
        out_specs=pl.BlockSpec((RV, 128), lambda r: (r, 0)),
        out_shape=jax.ShapeDtypeStruct((NV, 128), jnp.float32),
    )(xv, W1v)


def _mk_expand_sel():
    sel = np.zeros((128, 1024), np.float32)
    for j in range(8):
        for cc in range(128):
            sel[8 * (2 * j + cc // 64), j * 128 + cc] = 1.0
    return jnp.asarray(sel)


def _tcD_body(d0_ref, d1_ref, sel_ref, disx_ref):
    fd = lax.rsqrt(1.0 + d0_ref[...] + d1_ref[...])
    disx_ref[...] = jnp.dot(fd, sel_ref[...],
                            preferred_element_type=jnp.float32,
                            precision=lax.Precision.HIGHEST)


def _tcD(d0v, d1v, sel):
    return pl.pallas_call(
        _tcD_body,
        grid=(1,),
        in_specs=[
            pl.BlockSpec((DV, 128), lambda r: (0, 0)),
            pl.BlockSpec((DV, 128), lambda r: (0, 0)),
            pl.BlockSpec((128, 1024), lambda r: (0, 0)),
        ],
        out_specs=pl.BlockSpec((DV, 1024), lambda r: (0, 0)),
        out_shape=jax.ShapeDtypeStruct((DV, 1024), jnp.float32),
    )(d0v, d1v, sel)


def _tcA2_body(xw_ref, disv_ref, y1_ref):
    y1_ref[...] = xw_ref[...] * disv_ref[...]


def _tcA2(xwv, disv):
    return pl.pallas_call(
        _tcA2_body,
        grid=(GRID,),
        in_specs=[
            pl.BlockSpec((RV, 128), lambda r: (r, 0)),
            pl.BlockSpec((RV, 128), lambda r: (r, 0)),
        ],
        out_specs=pl.BlockSpec((RV, 128), lambda r: (r, 0)),
        out_shape=jax.ShapeDtypeStruct((NV, 128), jnp.float32),
    )(xwv, disv)


def _tcB_body(y1_ref, a0_ref, a1_ref, disv_ref, w2v_ref, b1v_ref, y2_ref):
    disv = disv_ref[...]
    tv = (a0_ref[...] + a1_ref[...] + y1_ref[...]) * disv + b1v_ref[...]
    hv = jnp.maximum(tv, 0.0)
    y2 = jnp.dot(hv, w2v_ref[...], preferred_element_type=jnp.float32)
    y2_ref[...] = y2 * disv


def _tcB(y1v, a0v, a1v, disv, W2v, b1v):
    return pl.pallas_call(
        _tcB_body,
        grid=(GRID,),
        in_specs=[
            pl.BlockSpec((RV, 128), lambda r: (r, 0)),
            pl.BlockSpec((RV, 128), lambda r: (r, 0)),
            pl.BlockSpec((RV, 128), lambda r: (r, 0)),
            pl.BlockSpec((RV, 128), lambda r: (r, 0)),
            pl.BlockSpec((128, 128), lambda r: (0, 0)),
            pl.BlockSpec((1, 128), lambda r: (0, 0)),
        ],
        out_specs=pl.BlockSpec((RV, 128), lambda r: (r, 0)),
        out_shape=jax.ShapeDtypeStruct((NV, 128), jnp.float32),
    )(y1v, a0v, a1v, disv, W2v, b1v)


def _tcC_body(y2_ref, a0_ref, a1_ref, disv_ref, b2v_ref, wlv_ref, blv_ref,
              q_ref):
    tv = ((a0_ref[...] + a1_ref[...] + y2_ref[...]) * disv_ref[...]
          + b2v_ref[...])
    hv = jnp.maximum(tv, 0.0)
    q_ref[...] = jnp.dot(hv, wlv_ref[...],
                         preferred_element_type=jnp.float32) + blv_ref[...]


def _tcC(y2v, a0v, a1v, disv, b2v, Wlv, blv):
    return pl.pallas_call(
        _tcC_body,
        grid=(GRID,),
        in_specs=[
            pl.BlockSpec((RV, 128), lambda r: (r, 0)),
            pl.BlockSpec((RV, 128), lambda r: (r, 0)),
            pl.BlockSpec((RV, 128), lambda r: (r, 0)),
            pl.BlockSpec((RV, 128), lambda r: (r, 0)),
            pl.BlockSpec((1, 128), lambda r: (0, 0)),
            pl.BlockSpec((128, 2 * OUT_CH), lambda r: (0, 0)),
            pl.BlockSpec((1, 2 * OUT_CH), lambda r: (0, 0)),
        ],
        out_specs=pl.BlockSpec((RV, 2 * OUT_CH), lambda r: (r, 0)),
        out_shape=jax.ShapeDtypeStruct((NV, 2 * OUT_CH), jnp.float32),
    )(y2v, a0v, a1v, disv, b2v, Wlv, blv)


def _blockdiag2(W):
    i, o = W.shape
    z = jnp.zeros((i, o), W.dtype)
    return jnp.concatenate(
        [jnp.concatenate([W, z], axis=1), jnp.concatenate([z, W], axis=1)],
        axis=0)


def kernel(x, edge_index, W1, b1, W2, b2, Wl, bl):
    src = edge_index[0].reshape(NW, NCH, K)
    dst = edge_index[1].reshape(NW, NCH, K)
    ones8 = jnp.ones((K, 8), jnp.float32)
    z8 = jnp.zeros((N, 8), jnp.float32)
    z64 = jnp.zeros((N, HID), jnp.float32)
    sel = _mk_expand_sel()
    b1v = jnp.concatenate([b1, b1]).reshape(1, 128)
    b2v = jnp.concatenate([b2, b2]).reshape(1, 128)
    blv = jnp.concatenate([bl, bl]).reshape(1, 2 * OUT_CH)
    W1v = _blockdiag2(W1)
    W2v = _blockdiag2(W2)
    Wlv = _blockdiag2(Wl)

    degp0, degp1 = _deg_sc(dst, ones8, z8)
    xwv = _tcA1(x.reshape(NV, 2 * IN_CH), W1v)
    disx = _tcD(degp0.reshape(DV, 128), degp1.reshape(DV, 128), sel)
    disv = disx.reshape(NV, 128)
    y1v = _tcA2(xwv, disv)
    a10, a11 = _agg_sc(src, dst, y1v.reshape(N, HID), z64)
    y2v = _tcB(y1v, a10.reshape(NV, 128), a11.reshape(NV, 128), disv,
               W2v, b1v)
    a20, a21 = _agg_sc(src, dst, y2v.reshape(N, HID), z64)
    qv = _tcC(y2v, a20.reshape(NV, 128), a21.reshape(NV, 128), disv,
              b2v, Wlv, blv)
    return qv.reshape(N, OUT_CH)


# speedup vs baseline: 49.1609x; 1.0929x over previous
import functools

import jax
import jax.numpy as jnp
import numpy as np
from jax import lax
from jax.experimental import pallas as pl
from jax.experimental.pallas import tpu as pltpu
from jax.experimental.pallas import tpu_sc as plsc

N = 10000
E = 320000
IN_CH = 128
HID = 64
OUT_CH = 2

NC = 2
NS = 16
NW = NC * NS
EPT = E // NW
K = 80
NCH = EPT // K
RPT = 624
TAIL = N - NS * RPT
TAIL_OFF = NS * RPT

NV = N * HID // 128
DV = N * 8 // 128

_mesh = plsc.VectorSubcoreMesh(core_axis_name="c", subcore_axis_name="s")


def _striped(s, copy_fn):
    copy_fn(s * RPT, RPT)

    @pl.when(s == 0)
    def _():
        copy_fn(TAIL_OFF, TAIL)


KC = 128
CHT = 2500
CPT = 78
XTRA = CHT - NW * CPT


@functools.partial(
    pl.kernel,
    mesh=_mesh,
    out_type=[
        jax.ShapeDtypeStruct((N, 8), jnp.float32),
        jax.ShapeDtypeStruct((N, 8), jnp.float32),
    ],
    scratch_types=[
        pltpu.VMEM((2 * CPT, KC), jnp.int32),
        pltpu.VMEM((2, KC), jnp.int32),
        pltpu.VMEM((KC, 8), jnp.float32),
        [pltpu.SemaphoreType.DMA] * 6,
        pltpu.VMEM_SHARED((N, 8), jnp.float32),
    ],
    compiler_params=pltpu.CompilerParams(use_tc_tiling_on_sc=False),
)
def _deg_sc(eit_hbm, ones_hbm, zeros_hbm, out0_hbm, out1_hbm, idx_v, xidx_v,
            ones_v, sems, acc_sh):
    c = lax.axis_index("c")
    s = lax.axis_index("s")
    wid = s * NC + c
    pltpu.sync_copy(eit_hbm.at[pl.ds(wid * 2 * CPT, 2 * CPT)], idx_v)
    pltpu.sync_copy(ones_hbm, ones_v)
    _striped(s, lambda off, n: pltpu.sync_copy(
        zeros_hbm.at[pl.ds(off, n)], acc_sh.at[pl.ds(off, n)]))
    plsc.subcore_barrier()

    def _scat(i, b):
        pltpu.async_copy(ones_v, acc_sh.at[idx_v.at[2 * i + 1]], sems[b],
                         add=True)

    def _wait(i, b):
        pltpu.make_async_copy(ones_v, acc_sh.at[idx_v.at[2 * i + 1]],
                              sems[b]).wait()

    for b in range(6):
        _scat(b, b)

    def body(j, carry):
        i = 6 * j
        for b in range(6):
            _wait(i - 6 + b, b)
            _scat(i + b, b)
        return carry

    lax.fori_loop(1, CPT // 6, body, 0)
    for b in range(6):
        _wait(CPT - 6 + b, b)

    @pl.when(wid < XTRA)
    def _():
        pltpu.sync_copy(eit_hbm.at[pl.ds(2 * (NW * CPT + wid), 2)], xidx_v)
        pltpu.sync_copy(ones_v, acc_sh.at[xidx_v.at[1]], add=True)

    plsc.subcore_barrier()

    @pl.when(c == 0)
    def _():
        _striped(s, lambda off, n: pltpu.sync_copy(
            acc_sh.at[pl.ds(off, n)], out0_hbm.at[pl.ds(off, n)]))

    @pl.when(c == 1)
    def _():
        _striped(s, lambda off, n: pltpu.sync_copy(
            acc_sh.at[pl.ds(off, n)], out1_hbm.at[pl.ds(off, n)]))


@functools.partial(
    pl.kernel,
    mesh=_mesh,
    out_type=[
        jax.ShapeDtypeStruct((N, HID), jnp.float32),
        jax.ShapeDtypeStruct((N, HID), jnp.float32),
    ],
    scratch_types=[
        pltpu.VMEM((2 * CPT, KC), jnp.int32),
        pltpu.VMEM((2, KC), jnp.int32),
        [pltpu.VMEM((KC, HID), jnp.float32)] * 6,
        [pltpu.SemaphoreType.DMA] * 6,
        [pltpu.SemaphoreType.DMA] * 6,
        pltpu.VMEM_SHARED((N, HID), jnp.float32),
    ],
    compiler_params=pltpu.CompilerParams(use_tc_tiling_on_sc=False),
)
def _agg_sc(eit_hbm, y_hbm, zeros_hbm, out0_hbm, out1_hbm,
            idx_v, xidx_v, bufs, gsems, ssems, acc_sh):
    c = lax.axis_index("c")
    s = lax.axis_index("s")
    wid = s * NC + c
    pltpu.sync_copy(eit_hbm.at[pl.ds(wid * 2 * CPT, 2 * CPT)], idx_v)
    _striped(s, lambda off, n: pltpu.sync_copy(
        zeros_hbm.at[pl.ds(off, n)], acc_sh.at[pl.ds(off, n)]))
    plsc.subcore_barrier()

    RING = 6

    def _gather(i, b):
        pltpu.async_copy(y_hbm.at[idx_v.at[2 * i]], bufs[b], gsems[b])

    def _wait_gather(i, b):
        pltpu.make_async_copy(y_hbm.at[idx_v.at[2 * i]], bufs[b],
                              gsems[b]).wait()

    def _scatter(i, b):
        pltpu.async_copy(bufs[b], acc_sh.at[idx_v.at[2 * i + 1]], ssems[b],
                         add=True)

    def _wait_scatter(i, b):
        pltpu.make_async_copy(bufs[b], acc_sh.at[idx_v.at[2 * i + 1]],
                              ssems[b]).wait()

    def _step(i, b, guards):
        _wait_gather(i, b)
        _scatter(i, b)
        b2 = (b + 2) % RING
        if guards:
            def _pf():
                _wait_scatter(i - 4, b2)
                _gather(i + 2, b2)
            if guards == "tail":
                pl.when(i + 2 < CPT)(_pf)
            else:
                _pf()
        else:
            _gather(i + 2, b2)

    _gather(0, 0)
    _gather(1, 1)
    _step(0, 0, None)
    _step(1, 1, None)
    _step(2, 2, None)
    _step(3, 3, None)
    _step(4, 4, "steady")
    _step(5, 5, "steady")

    def body(j, carry):
        i = RING * j
        _step(i, 0, "steady")
        _step(i + 1, 1, "steady")
        _step(i + 2, 2, "steady")
        _step(i + 3, 3, "steady")
        _step(i + 4, 4, "tail")
        _step(i + 5, 5, "tail")
        return carry

    lax.fori_loop(1, CPT // RING, body, 0)

    _wait_scatter(72, 0)
    _wait_scatter(73, 1)
    _wait_scatter(74, 2)
    _wait_scatter(75, 3)
    _wait_scatter(76, 4)
    _wait_scatter(77, 5)

    @pl.when(wid < XTRA)
    def _():
        pltpu.sync_copy(eit_hbm.at[pl.ds(2 * (NW * CPT + wid), 2)], xidx_v)
        pltpu.async_copy(y_hbm.at[xidx_v.at[0]], bufs[0],
                         gsems[0]).wait()
        pltpu.sync_copy(bufs[0], acc_sh.at[xidx_v.at[1]], add=True)

    plsc.subcore_barrier()

    @pl.when(c == 0)
    def _():
        _striped(s, lambda off, n: pltpu.sync_copy(
            acc_sh.at[pl.ds(off, n)], out0_hbm.at[pl.ds(off, n)]))

    @pl.when(c == 1)
    def _():
        _striped(s, lambda off, n: pltpu.sync_copy(
            acc_sh.at[pl.ds(off, n)], out1_hbm.at[pl.ds(off, n)]))


RV = 1000
GRID = NV // RV


def _tcA1_body(x_ref, w1_ref, xw_ref):
    xw_ref[...] = jnp.dot(x_ref[...], w1_ref[...],
                          preferred_element_type=jnp.float32)


def _tcA1(xv, W1v):
    return pl.pallas_call(
        _tcA1_body,
        grid=(GRID,),
        in_specs=[
            pl.BlockSpec((RV, 2 * IN_CH), lambda r: (r, 0)),
            pl.BlockSpec((2 * IN_CH, 128), lambda r: (0, 0)),
        ],
        out_specs=pl.BlockSpec((RV, 128), lambda r: (r, 0)),
        out_shape=jax.ShapeDtypeStruct((NV, 128), jnp.float32),
    )(xv, W1v)


def _mk_expand_sel():
    sel = np.zeros((128, 1024), np.float32)
    for j in range(8):
        for cc in range(128):
            sel[8 * (2 * j + cc // 64), j * 128 + cc] = 1.0
    return jnp.asarray(sel)


def _tcD_body(d0_ref, d1_ref, sel_ref, disx_ref):
    fd = lax.rsqrt(1.0 + d0_ref[...] + d1_ref[...])
    disx_ref[...] = jnp.dot(fd, sel_ref[...],
                            preferred_element_type=jnp.float32,
                            precision=lax.Precision.HIGHEST)


def _tcD(d0v, d1v, sel):
    return pl.pallas_call(
        _tcD_body,
        grid=(1,),
        in_specs=[
            pl.BlockSpec((DV, 128), lambda r: (0, 0)),
            pl.BlockSpec((DV, 128), lambda r: (0, 0)),
            pl.BlockSpec((128, 1024), lambda r: (0, 0)),
        ],
        out_specs=pl.BlockSpec((DV, 1024), lambda r: (0, 0)),
        out_shape=jax.ShapeDtypeStruct((DV, 1024), jnp.float32),
    )(d0v, d1v, sel)


def _tcA2_body(xw_ref, disv_ref, y1_ref):
    y1_ref[...] = xw_ref[...] * disv_ref[...]


def _tcA2(xwv, disv):
    return pl.pallas_call(
        _tcA2_body,
        grid=(GRID,),
        in_specs=[
            pl.BlockSpec((RV, 128), lambda r: (r, 0)),
            pl.BlockSpec((RV, 128), lambda r: (r, 0)),
        ],
        out_specs=pl.BlockSpec((RV, 128), lambda r: (r, 0)),
        out_shape=jax.ShapeDtypeStruct((NV, 128), jnp.float32),
    )(xwv, disv)


def _tcB_body(y1_ref, a0_ref, a1_ref, disv_ref, w2v_ref, b1v_ref, y2_ref):
    disv = disv_ref[...]
    tv = (a0_ref[...] + a1_ref[...] + y1_ref[...]) * disv + b1v_ref[...]
    hv = jnp.maximum(tv, 0.0)
    y2 = jnp.dot(hv, w2v_ref[...], preferred_element_type=jnp.float32)
    y2_ref[...] = y2 * disv


def _tcB(y1v, a0v, a1v, disv, W2v, b1v):
    return pl.pallas_call(
        _tcB_body,
        grid=(GRID,),
        in_specs=[
            pl.BlockSpec((RV, 128), lambda r: (r, 0)),
            pl.BlockSpec((RV, 128), lambda r: (r, 0)),
            pl.BlockSpec((RV, 128), lambda r: (r, 0)),
            pl.BlockSpec((RV, 128), lambda r: (r, 0)),
            pl.BlockSpec((128, 128), lambda r: (0, 0)),
            pl.BlockSpec((1, 128), lambda r: (0, 0)),
        ],
        out_specs=pl.BlockSpec((RV, 128), lambda r: (r, 0)),
        out_shape=jax.ShapeDtypeStruct((NV, 128), jnp.float32),
    )(y1v, a0v, a1v, disv, W2v, b1v)


def _tcC_body(y2_ref, a0_ref, a1_ref, disv_ref, b2v_ref, wlv_ref, blv_ref,
              q_ref):
    tv = ((a0_ref[...] + a1_ref[...] + y2_ref[...]) * disv_ref[...]
          + b2v_ref[...])
    hv = jnp.maximum(tv, 0.0)
    q_ref[...] = jnp.dot(hv, wlv_ref[...],
                         preferred_element_type=jnp.float32) + blv_ref[...]


def _tcC(y2v, a0v, a1v, disv, b2v, Wlv, blv):
    return pl.pallas_call(
        _tcC_body,
        grid=(GRID,),
        in_specs=[
            pl.BlockSpec((RV, 128), lambda r: (r, 0)),
            pl.BlockSpec((RV, 128), lambda r: (r, 0)),
            pl.BlockSpec((RV, 128), lambda r: (r, 0)),
            pl.BlockSpec((RV, 128), lambda r: (r, 0)),
            pl.BlockSpec((1, 128), lambda r: (0, 0)),
            pl.BlockSpec((128, 2 * OUT_CH), lambda r: (0, 0)),
            pl.BlockSpec((1, 2 * OUT_CH), lambda r: (0, 0)),
        ],
        out_specs=pl.BlockSpec((RV, 2 * OUT_CH), lambda r: (r, 0)),
        out_shape=jax.ShapeDtypeStruct((NV, 2 * OUT_CH), jnp.float32),
    )(y2v, a0v, a1v, disv, b2v, Wlv, blv)


def _blockdiag2(W):
    i, o = W.shape
    z = jnp.zeros((i, o), W.dtype)
    return jnp.concatenate(
        [jnp.concatenate([W, z], axis=1), jnp.concatenate([z, W], axis=1)],
        axis=0)


def kernel(x, edge_index, W1, b1, W2, b2, Wl, bl):
    eit = jnp.transpose(edge_index.reshape(2, CHT, KC),
                        (1, 0, 2)).reshape(2 * CHT, KC)
    ones8 = jnp.ones((KC, 8), jnp.float32)
    z8 = jnp.zeros((N, 8), jnp.float32)
    z64 = jnp.zeros((N, HID), jnp.float32)
    sel = _mk_expand_sel()
    b1v = jnp.concatenate([b1, b1]).reshape(1, 128)
    b2v = jnp.concatenate([b2, b2]).reshape(1, 128)
    blv = jnp.concatenate([bl, bl]).reshape(1, 2 * OUT_CH)
    W1v = _blockdiag2(W1)
    W2v = _blockdiag2(W2)
    Wlv = _blockdiag2(Wl)

    degp0, degp1 = _deg_sc(eit, ones8, z8)
    xwv = _tcA1(x.reshape(NV, 2 * IN_CH), W1v)
    disx = _tcD(degp0.reshape(DV, 128), degp1.reshape(DV, 128), sel)
    disv = disx.reshape(NV, 128)
    y1v = _tcA2(xwv, disv)
    a10, a11 = _agg_sc(eit, y1v.reshape(N, HID), z64)
    y2v = _tcB(y1v, a10.reshape(NV, 128), a11.reshape(NV, 128), disv,
               W2v, b1v)
    a20, a21 = _agg_sc(eit, y2v.reshape(N, HID), z64)
    qv = _tcC(y2v, a20.reshape(NV, 128), a21.reshape(NV, 128), disv,
              b2v, Wlv, blv)
    return qv.reshape(N, OUT_CH)


# speedup vs baseline: 52.1286x; 1.0604x over previous
import functools

import jax
import jax.numpy as jnp
import numpy as np
from jax import lax
from jax.experimental import pallas as pl
from jax.experimental.pallas import tpu as pltpu
from jax.experimental.pallas import tpu_sc as plsc

N = 10000
E = 320000
IN_CH = 128
HID = 64
OUT_CH = 2

NC = 2
NS = 16
NW = NC * NS
EPT = E // NW
K = 80
NCH = EPT // K
RPT = 624
TAIL = N - NS * RPT
TAIL_OFF = NS * RPT

NV = N * HID // 128
DV = N * 8 // 128

_mesh = plsc.VectorSubcoreMesh(core_axis_name="c", subcore_axis_name="s")


def _striped(s, copy_fn):
    copy_fn(s * RPT, RPT)

    @pl.when(s == 0)
    def _():
        copy_fn(TAIL_OFF, TAIL)


KC = 128
CHT = 2500
CPT = 78
XTRA = CHT - NW * CPT


@functools.partial(
    pl.kernel,
    mesh=_mesh,
    out_type=[
        jax.ShapeDtypeStruct((N, 8), jnp.float32),
        jax.ShapeDtypeStruct((N, 8), jnp.float32),
    ],
    scratch_types=[
        pltpu.VMEM((2 * CPT, KC), jnp.int32),
        pltpu.VMEM((2, KC), jnp.int32),
        pltpu.VMEM((KC, 8), jnp.float32),
        [pltpu.SemaphoreType.DMA] * 6,
        pltpu.VMEM_SHARED((N, 8), jnp.float32),
    ],
    compiler_params=pltpu.CompilerParams(use_tc_tiling_on_sc=False),
)
def _deg_sc(eit_hbm, ones_hbm, zeros_hbm, out0_hbm, out1_hbm, idx_v, xidx_v,
            ones_v, sems, acc_sh):
    c = lax.axis_index("c")
    s = lax.axis_index("s")
    wid = s * NC + c
    pltpu.sync_copy(eit_hbm.at[pl.ds(wid * 2 * CPT, 2 * CPT)], idx_v)
    pltpu.sync_copy(ones_hbm, ones_v)
    _striped(s, lambda off, n: pltpu.sync_copy(
        zeros_hbm.at[pl.ds(off, n)], acc_sh.at[pl.ds(off, n)]))
    plsc.subcore_barrier()

    def _scat(i, b):
        pltpu.async_copy(ones_v, acc_sh.at[idx_v.at[2 * i + 1]], sems[b],
                         add=True)

    def _wait(i, b):
        pltpu.make_async_copy(ones_v, acc_sh.at[idx_v.at[2 * i + 1]],
                              sems[b]).wait()

    for b in range(6):
        _scat(b, b)

    def body(j, carry):
        i = 6 * j
        for b in range(6):
            _wait(i - 6 + b, b)
            _scat(i + b, b)
        return carry

    lax.fori_loop(1, CPT // 6, body, 0)
    for b in range(6):
        _wait(CPT - 6 + b, b)

    @pl.when(wid < XTRA)
    def _():
        pltpu.sync_copy(eit_hbm.at[pl.ds(2 * (NW * CPT + wid), 2)], xidx_v)
        pltpu.sync_copy(ones_v, acc_sh.at[xidx_v.at[1]], add=True)

    plsc.subcore_barrier()

    @pl.when(c == 0)
    def _():
        _striped(s, lambda off, n: pltpu.sync_copy(
            acc_sh.at[pl.ds(off, n)], out0_hbm.at[pl.ds(off, n)]))

    @pl.when(c == 1)
    def _():
        _striped(s, lambda off, n: pltpu.sync_copy(
            acc_sh.at[pl.ds(off, n)], out1_hbm.at[pl.ds(off, n)]))


@functools.partial(
    pl.kernel,
    mesh=_mesh,
    out_type=[
        jax.ShapeDtypeStruct((N, HID), jnp.float32),
        jax.ShapeDtypeStruct((N, HID), jnp.float32),
    ],
    scratch_types=[
        pltpu.VMEM((2 * CPT, KC), jnp.int32),
        pltpu.VMEM((2, KC), jnp.int32),
        [pltpu.VMEM((KC, HID), jnp.float32)] * 6,
        [pltpu.SemaphoreType.DMA] * 6,
        [pltpu.SemaphoreType.DMA] * 6,
        pltpu.VMEM_SHARED((N, HID), jnp.float32),
    ],
    compiler_params=pltpu.CompilerParams(use_tc_tiling_on_sc=False),
)
def _agg_sc(eit_hbm, y_hbm, zeros_hbm, out0_hbm, out1_hbm,
            idx_v, xidx_v, bufs, gsems, ssems, acc_sh):
    c = lax.axis_index("c")
    s = lax.axis_index("s")
    wid = s * NC + c
    pltpu.sync_copy(eit_hbm.at[pl.ds(wid * 2 * CPT, 2 * CPT)], idx_v)
    _striped(s, lambda off, n: pltpu.sync_copy(
        zeros_hbm.at[pl.ds(off, n)], acc_sh.at[pl.ds(off, n)]))
    plsc.subcore_barrier()

    RING = 6

    def _gather(i, b):
        pltpu.async_copy(y_hbm.at[idx_v.at[2 * i]], bufs[b], gsems[b])

    def _wait_gather(i, b):
        pltpu.make_async_copy(y_hbm.at[idx_v.at[2 * i]], bufs[b],
                              gsems[b]).wait()

    def _scatter(i, b):
        pltpu.async_copy(bufs[b], acc_sh.at[idx_v.at[2 * i + 1]], ssems[b],
                         add=True)

    def _wait_scatter(i, b):
        pltpu.make_async_copy(bufs[b], acc_sh.at[idx_v.at[2 * i + 1]],
                              ssems[b]).wait()

    def _step(i, b, guards):
        _wait_gather(i, b)
        _scatter(i, b)
        b3 = (b + 3) % RING
        if guards:
            def _pf():
                _wait_scatter(i - 3, b3)
                _gather(i + 3, b3)
            if guards == "tail":
                pl.when(i + 3 < CPT)(_pf)
            else:
                _pf()
        else:
            _gather(i + 3, b3)

    _gather(0, 0)
    _gather(1, 1)
    _gather(2, 2)
    _step(0, 0, None)
    _step(1, 1, None)
    _step(2, 2, None)
    _step(3, 3, "steady")
    _step(4, 4, "steady")
    _step(5, 5, "steady")

    def body(j, carry):
        i = RING * j
        _step(i, 0, "steady")
        _step(i + 1, 1, "steady")
        _step(i + 2, 2, "steady")
        _step(i + 3, 3, "tail")
        _step(i + 4, 4, "tail")
        _step(i + 5, 5, "tail")
        return carry

    lax.fori_loop(1, CPT // RING, body, 0)

    _wait_scatter(72, 0)
    _wait_scatter(73, 1)
    _wait_scatter(74, 2)
    _wait_scatter(75, 3)
    _wait_scatter(76, 4)
    _wait_scatter(77, 5)

    @pl.when(wid < XTRA)
    def _():
        pltpu.sync_copy(eit_hbm.at[pl.ds(2 * (NW * CPT + wid), 2)], xidx_v)
        pltpu.async_copy(y_hbm.at[xidx_v.at[0]], bufs[0],
                         gsems[0]).wait()
        pltpu.sync_copy(bufs[0], acc_sh.at[xidx_v.at[1]], add=True)

    plsc.subcore_barrier()

    @pl.when(c == 0)
    def _():
        _striped(s, lambda off, n: pltpu.sync_copy(
            acc_sh.at[pl.ds(off, n)], out0_hbm.at[pl.ds(off, n)]))

    @pl.when(c == 1)
    def _():
        _striped(s, lambda off, n: pltpu.sync_copy(
            acc_sh.at[pl.ds(off, n)], out1_hbm.at[pl.ds(off, n)]))


RV = 1000
GRID = NV // RV


def _tcA1_body(x_ref, w1_ref, xw_ref):
    xw_ref[...] = jnp.dot(x_ref[...], w1_ref[...],
                          preferred_element_type=jnp.float32)


def _tcA1(xv, W1v):
    return pl.pallas_call(
        _tcA1_body,
        grid=(GRID,),
        in_specs=[
            pl.BlockSpec((RV, 2 * IN_CH), lambda r: (r, 0)),
            pl.BlockSpec((2 * IN_CH, 128), lambda r: (0, 0)),
        ],
        out_specs=pl.BlockSpec((RV, 128), lambda r: (r, 0)),
        out_shape=jax.ShapeDtypeStruct((NV, 128), jnp.float32),
    )(xv, W1v)


def _mk_expand_sel():
    sel = np.zeros((128, 1024), np.float32)
    for j in range(8):
        for cc in range(128):
            sel[8 * (2 * j + cc // 64), j * 128 + cc] = 1.0
    return jnp.asarray(sel)


def _tcD_body(d0_ref, d1_ref, sel_ref, disx_ref):
    fd = lax.rsqrt(1.0 + d0_ref[...] + d1_ref[...])
    for j in range(8):
        disx_ref[:, j, :] = jnp.dot(fd, sel_ref[:, j * 128:(j + 1) * 128],
                                    preferred_element_type=jnp.float32,
                                    precision=lax.Precision.HIGHEST)


def _tcD(d0v, d1v, sel):
    return pl.pallas_call(
        _tcD_body,
        grid=(1,),
        in_specs=[
            pl.BlockSpec((DV, 128), lambda r: (0, 0)),
            pl.BlockSpec((DV, 128), lambda r: (0, 0)),
            pl.BlockSpec((128, 1024), lambda r: (0, 0)),
        ],
        out_specs=pl.BlockSpec((DV, 8, 128), lambda r: (0, 0, 0)),
        out_shape=jax.ShapeDtypeStruct((DV, 8, 128), jnp.float32),
    )(d0v, d1v, sel)


def _tcA2_body(xw_ref, disv_ref, y1_ref):
    y1_ref[...] = xw_ref[...] * disv_ref[...]


def _tcA2(xwv, disv):
    return pl.pallas_call(
        _tcA2_body,
        grid=(GRID,),
        in_specs=[
            pl.BlockSpec((RV, 128), lambda r: (r, 0)),
            pl.BlockSpec((RV, 128), lambda r: (r, 0)),
        ],
        out_specs=pl.BlockSpec((RV, 128), lambda r: (r, 0)),
        out_shape=jax.ShapeDtypeStruct((NV, 128), jnp.float32),
    )(xwv, disv)


def _tcB_body(y1_ref, a0_ref, a1_ref, disv_ref, w2v_ref, b1v_ref, y2_ref):
    disv = disv_ref[...]
    tv = (a0_ref[...] + a1_ref[...] + y1_ref[...]) * disv + b1v_ref[...]
    hv = jnp.maximum(tv, 0.0)
    y2 = jnp.dot(hv, w2v_ref[...], preferred_element_type=jnp.float32)
    y2_ref[...] = y2 * disv


def _tcB(y1v, a0v, a1v, disv, W2v, b1v):
    return pl.pallas_call(
        _tcB_body,
        grid=(GRID,),
        in_specs=[
            pl.BlockSpec((RV, 128), lambda r: (r, 0)),
            pl.BlockSpec((RV, 128), lambda r: (r, 0)),
            pl.BlockSpec((RV, 128), lambda r: (r, 0)),
            pl.BlockSpec((RV, 128), lambda r: (r, 0)),
            pl.BlockSpec((128, 128), lambda r: (0, 0)),
            pl.BlockSpec((1, 128), lambda r: (0, 0)),
        ],
        out_specs=pl.BlockSpec((RV, 128), lambda r: (r, 0)),
        out_shape=jax.ShapeDtypeStruct((NV, 128), jnp.float32),
    )(y1v, a0v, a1v, disv, W2v, b1v)


def _tcC_body(y2_ref, a0_ref, a1_ref, disv_ref, b2v_ref, wlv_ref, blv_ref,
              q_ref):
    tv = ((a0_ref[...] + a1_ref[...] + y2_ref[...]) * disv_ref[...]
          + b2v_ref[...])
    hv = jnp.maximum(tv, 0.0)
    q_ref[...] = jnp.dot(hv, wlv_ref[...],
                         preferred_element_type=jnp.float32) + blv_ref[...]


def _tcC(y2v, a0v, a1v, disv, b2v, Wlv, blv):
    return pl.pallas_call(
        _tcC_body,
        grid=(GRID,),
        in_specs=[
            pl.BlockSpec((RV, 128), lambda r: (r, 0)),
            pl.BlockSpec((RV, 128), lambda r: (r, 0)),
            pl.BlockSpec((RV, 128), lambda r: (r, 0)),
            pl.BlockSpec((RV, 128), lambda r: (r, 0)),
            pl.BlockSpec((1, 128), lambda r: (0, 0)),
            pl.BlockSpec((128, 2 * OUT_CH), lambda r: (0, 0)),
            pl.BlockSpec((1, 2 * OUT_CH), lambda r: (0, 0)),
        ],
        out_specs=pl.BlockSpec((RV, 2 * OUT_CH), lambda r: (r, 0)),
        out_shape=jax.ShapeDtypeStruct((NV, 2 * OUT_CH), jnp.float32),
    )(y2v, a0v, a1v, disv, b2v, Wlv, blv)


def _blockdiag2(W):
    i, o = W.shape
    z = jnp.zeros((i, o), W.dtype)
    return jnp.concatenate(
        [jnp.concatenate([W, z], axis=1), jnp.concatenate([z, W], axis=1)],
        axis=0)


def kernel(x, edge_index, W1, b1, W2, b2, Wl, bl):
    eit = jnp.transpose(edge_index.reshape(2, CHT, KC),
                        (1, 0, 2)).reshape(2 * CHT, KC)
    ones8 = jnp.ones((KC, 8), jnp.float32)
    z8 = jnp.zeros((N, 8), jnp.float32)
    z64 = jnp.zeros((N, HID), jnp.float32)
    sel = _mk_expand_sel()
    b1v = jnp.concatenate([b1, b1]).reshape(1, 128)
    b2v = jnp.concatenate([b2, b2]).reshape(1, 128)
    blv = jnp.concatenate([bl, bl]).reshape(1, 2 * OUT_CH)
    W1v = _blockdiag2(W1)
    W2v = _blockdiag2(W2)
    Wlv = _blockdiag2(Wl)

    degp0, degp1 = _deg_sc(eit, ones8, z8)
    xwv = _tcA1(x.reshape(NV, 2 * IN_CH), W1v)
    disx = _tcD(degp0.reshape(DV, 128), degp1.reshape(DV, 128), sel)
    disv = disx.reshape(NV, 128)
    y1v = _tcA2(xwv, disv)
    a10, a11 = _agg_sc(eit, y1v.reshape(N, HID), z64)
    y2v = _tcB(y1v, a10.reshape(NV, 128), a11.reshape(NV, 128), disv,
               W2v, b1v)
    a20, a21 = _agg_sc(eit, y2v.reshape(N, HID), z64)
    qv = _tcC(y2v, a20.reshape(NV, 128), a21.reshape(NV, 128), disv,
              b2v, Wlv, blv)
    return qv.reshape(N, OUT_CH)
